# Initial kernel scaffold; baseline (speedup 1.0000x reference)
import jax
import jax.numpy as jnp
from jax.experimental import pallas as pl


def kernel(x, edge_type_idcs, edge_masks, self_weight, relation_weights):
    raise NotImplementedError("write your pallas kernel here")


# speedup vs baseline: 3.1145x; 3.1145x over previous
import functools

import jax
import jax.numpy as jnp
from jax import lax
from jax.experimental import pallas as pl
from jax.experimental.pallas import tpu as pltpu
from jax.experimental.pallas import tpu_sc as plsc

N_NODES = 10000
D = 128
N_REL = 8
E = 40000

W = 144
NC, NS = 1, 16
LANES = 16
CHUNK = 128
NCH = 20
EPT = CHUNK * NCH
EP = EPT * NS
REL_PER_CORE = N_REL // NC
ACC_ROWS = 10240
N_TRASH = ACC_ROWS - N_NODES
ZROWS = 128
ROWS_PER_TILE = ACC_ROWS // NS
WB = 624
WBC = 104
WB_TAIL = N_NODES - NS * WB


def _sc_body(x1_hbm, ep_hbm, zeros_hbm, out_hbm,
             idx_p, idx_s, idx_d, rows, acc):
    c = lax.axis_index("c")
    t = lax.axis_index("s")

    def rel_step(rl, carry):
        r = c * REL_PER_CORE + rl
        pltpu.sync_copy(zeros_hbm, rows)

        def zero_step(z, carry2):
            pltpu.sync_copy(
                rows, acc.at[pl.ds(t * ROWS_PER_TILE + z * ZROWS, ZROWS)])
            return carry2

        lax.fori_loop(0, ROWS_PER_TILE // ZROWS, zero_step, 0)
        pltpu.sync_copy(ep_hbm.at[r, t], idx_p)
        plsc.subcore_barrier()

        def chunk_step(j, carry2):
            def unpack_vec(k, carry3):
                v = idx_p[j, pl.ds(k * LANES, LANES)]
                idx_s[0, pl.ds(k * LANES, LANES)] = v & 0xFFFF
                idx_d[0, pl.ds(k * LANES, LANES)] = v >> 16
                return carry3

            lax.fori_loop(0, CHUNK // LANES, unpack_vec, 0)
            pltpu.sync_copy(x1_hbm.at[idx_s.at[0]], rows)
            pltpu.sync_copy(rows, acc.at[idx_d.at[0]], add=True)
            return carry2

        lax.fori_loop(0, NCH, chunk_step, 0)
        plsc.subcore_barrier()

        def wb_step(k, carry2):
            base = t * WB + k * WBC
            pltpu.sync_copy(acc.at[pl.ds(base, WBC)], rows.at[pl.ds(0, WBC)])
            pltpu.sync_copy(rows.at[pl.ds(0, WBC)],
                            out_hbm.at[pl.ds(r * N_NODES + base, WBC)])
            return carry2

        lax.fori_loop(0, WB // WBC, wb_step, 0)

        @pl.when(t == 0)
        def _():
            pltpu.sync_copy(acc.at[pl.ds(NS * WB, WB_TAIL)],
                            rows.at[pl.ds(0, WB_TAIL)])
            pltpu.sync_copy(
                rows.at[pl.ds(0, WB_TAIL)],
                out_hbm.at[pl.ds(r * N_NODES + NS * WB, WB_TAIL)])

        plsc.subcore_barrier()
        return carry

    lax.fori_loop(0, REL_PER_CORE, rel_step, 0)


def _sc_aggregate(x1, edges_packed, zeros):
    mesh = plsc.VectorSubcoreMesh(
        core_axis_name="c", subcore_axis_name="s", num_cores=NC)
    return pl.kernel(
        _sc_body,
        out_type=jax.ShapeDtypeStruct((N_REL * N_NODES, W), jnp.float32),
        mesh=mesh,
        scratch_types=[
            pltpu.VMEM((NCH, CHUNK), jnp.int32),
            pltpu.VMEM((1, CHUNK), jnp.int32),
            pltpu.VMEM((1, CHUNK), jnp.int32),
            pltpu.VMEM((CHUNK, W), jnp.float32),
            pltpu.VMEM_SHARED((ACC_ROWS, W), jnp.float32),
        ],
        compiler_params=pltpu.CompilerParams(use_tc_tiling_on_sc=False),
    )(x1, edges_packed, zeros)


def _tc_body(x_ref, acc_ref, w_ref, out_ref):
    xb = x_ref[...]
    out = jnp.dot(xb, w_ref[N_REL], preferred_element_type=jnp.float32)
    for r in range(N_REL):
        a = acc_ref[r]
        cnt = a[:, D:D + 1]
        agg = a[:, :D] / jnp.maximum(cnt, 1.0)
        out = out + jnp.dot(agg, w_ref[r], preferred_element_type=jnp.float32)
    out_ref[...] = out


def _tc_combine(x, acc3, w_all):
    blk = 400
    grid = (N_NODES // blk,)
    return pl.pallas_call(
        _tc_body,
        grid=grid,
        in_specs=[
            pl.BlockSpec((blk, D), lambda i: (i, 0)),
            pl.BlockSpec((N_REL, blk, W), lambda i: (0, i, 0)),
            pl.BlockSpec((N_REL + 1, D, D), lambda i: (0, 0, 0)),
        ],
        out_specs=pl.BlockSpec((blk, D), lambda i: (i, 0)),
        out_shape=jax.ShapeDtypeStruct((N_NODES, D), jnp.float32),
    )(x, acc3, w_all)


def kernel(x, edge_type_idcs, edge_masks, self_weight, relation_weights):
    x1 = jnp.concatenate(
        [x, jnp.ones((N_NODES, 1), jnp.float32),
         jnp.zeros((N_NODES, W - D - 1), jnp.float32)], axis=1)

    src = edge_type_idcs[:, 0, :].astype(jnp.int32)
    dst = edge_type_idcs[:, 1, :].astype(jnp.int32)
    trash = N_NODES + (jnp.arange(E, dtype=jnp.int32) % N_TRASH)
    dst = jnp.where(edge_masks, dst, trash[None, :])

    pad = EP - E
    pad_src = (jnp.arange(pad, dtype=jnp.int32) * 37) % N_NODES
    pad_dst = N_NODES + jnp.arange(pad, dtype=jnp.int32) % N_TRASH
    src_p = jnp.concatenate(
        [src, jnp.broadcast_to(pad_src, (N_REL, pad))], axis=1)
    dst_p = jnp.concatenate(
        [dst, jnp.broadcast_to(pad_dst, (N_REL, pad))], axis=1)
    packed = (src_p | (dst_p << 16)).reshape(N_REL, NS, NCH, CHUNK)

    zeros = jnp.zeros((ZROWS, W), jnp.float32)
    acc = _sc_aggregate(x1, packed, zeros)
    acc3 = acc.reshape(N_REL, N_NODES, W)

    w_all = jnp.concatenate([relation_weights, self_weight[None]], axis=0)
    return _tc_combine(x, acc3, w_all)
